# Initial kernel scaffold; baseline (speedup 1.0000x reference)
import jax
import jax.numpy as jnp
from jax.experimental import pallas as pl


def kernel(X, W, A_stack, B_stack, centroids):
    raise NotImplementedError("write your pallas kernel here")


# speedup vs baseline: 1.5496x; 1.5496x over previous
import jax
import jax.numpy as jnp
from jax.experimental import pallas as pl
from jax.experimental.pallas import tpu as pltpu

_B, _S, _D, _G = 32, 256, 768, 8
_QMAX = 127.0
_EPS = 1e-8


def _labels_kernel(x_ref, c_ref, lab_ref):
    b = pl.program_id(0)
    x = x_ref[0]
    m = jnp.mean(x, axis=0, keepdims=True)
    xc = x - m
    var = jnp.sum(xc * xc, axis=0, keepdims=True) / (_S - 1)
    stats = jnp.concatenate([m, jnp.sqrt(var)], axis=1)
    diff = stats - c_ref[...]
    d2 = jnp.sum(diff * diff, axis=1, keepdims=True)
    idx = jax.lax.broadcasted_iota(jnp.int32, (_G, 1), 0)
    lab = jnp.min(jnp.where(d2 == jnp.min(d2), idx, _G))
    lab_ref[b] = lab.astype(jnp.int32)


def _bw_kernel(lab_ref, b_ref, w_ref, bw_ref, bwmax_ref):
    g = pl.program_id(0)
    bw = jnp.dot(b_ref[0], w_ref[...], preferred_element_type=jnp.float32)
    bw_ref[0] = bw
    used = lab_ref[0] == g
    for i in range(1, _B):
        used = used | (lab_ref[i] == g)
    bwmax_ref[g] = jnp.where(used, jnp.max(jnp.abs(bw)), 0.0)


def _xa_kernel(lab_ref, x_ref, a_ref, xa_ref, xamax_ref):
    b = pl.program_id(0)
    xa = jnp.dot(x_ref[0], a_ref[0], preferred_element_type=jnp.float32)
    xa_ref[0] = xa
    m = jnp.max(jnp.abs(xa))

    @pl.when(b == 0)
    def _():
        xamax_ref[0, 0] = m

    @pl.when(b > 0)
    def _():
        xamax_ref[0, 0] = jnp.maximum(xamax_ref[0, 0], m)


def _final_kernel(lab_ref, xa_ref, bw_ref, xamax_ref, bwmax_ref, out_ref):
    sxa = jnp.maximum(xamax_ref[0, 0] / _QMAX, _EPS)
    bm = bwmax_ref[0]
    for g in range(1, _G):
        bm = jnp.maximum(bm, bwmax_ref[g])
    sbw = jnp.maximum(bm / _QMAX, _EPS)
    qxa = jnp.round(xa_ref[0] * (1.0 / sxa)).astype(jnp.bfloat16)
    qbw = jnp.round(bw_ref[0] * (1.0 / sbw)).astype(jnp.bfloat16)
    acc = jnp.dot(qxa, qbw, preferred_element_type=jnp.float32)
    out_ref[0] = acc * (sxa * sbw)


def kernel(X, W, A_stack, B_stack, centroids):
    labels = pl.pallas_call(
        _labels_kernel,
        grid=(_B,),
        in_specs=[
            pl.BlockSpec((1, _S, _D), lambda b: (b, 0, 0)),
            pl.BlockSpec((_G, 2 * _D), lambda b: (0, 0)),
        ],
        out_specs=pl.BlockSpec((_B,), lambda b: (0,), memory_space=pltpu.SMEM),
        out_shape=jax.ShapeDtypeStruct((_B,), jnp.int32),
    )(X, centroids)

    bw, bwmax = pl.pallas_call(
        _bw_kernel,
        grid_spec=pltpu.PrefetchScalarGridSpec(
            num_scalar_prefetch=1,
            grid=(_G,),
            in_specs=[
                pl.BlockSpec((1, _D, _D), lambda g, lab: (g, 0, 0)),
                pl.BlockSpec((_D, _D), lambda g, lab: (0, 0)),
            ],
            out_specs=[
                pl.BlockSpec((1, _D, _D), lambda g, lab: (g, 0, 0)),
                pl.BlockSpec((_G,), lambda g, lab: (0,), memory_space=pltpu.SMEM),
            ],
        ),
        out_shape=[
            jax.ShapeDtypeStruct((_G, _D, _D), jnp.float32),
            jax.ShapeDtypeStruct((_G,), jnp.float32),
        ],
    )(labels, B_stack, W)

    xa, xamax = pl.pallas_call(
        _xa_kernel,
        grid_spec=pltpu.PrefetchScalarGridSpec(
            num_scalar_prefetch=1,
            grid=(_B,),
            in_specs=[
                pl.BlockSpec((1, _S, _D), lambda b, lab: (b, 0, 0)),
                pl.BlockSpec((1, _D, _D), lambda b, lab: (lab[b], 0, 0)),
            ],
            out_specs=[
                pl.BlockSpec((1, _S, _D), lambda b, lab: (b, 0, 0)),
                pl.BlockSpec((1, 1), lambda b, lab: (0, 0), memory_space=pltpu.SMEM),
            ],
        ),
        out_shape=[
            jax.ShapeDtypeStruct((_B, _S, _D), jnp.float32),
            jax.ShapeDtypeStruct((1, 1), jnp.float32),
        ],
    )(labels, X, A_stack)

    out = pl.pallas_call(
        _final_kernel,
        grid_spec=pltpu.PrefetchScalarGridSpec(
            num_scalar_prefetch=1,
            grid=(_B,),
            in_specs=[
                pl.BlockSpec((1, _S, _D), lambda b, lab: (b, 0, 0)),
                pl.BlockSpec((1, _D, _D), lambda b, lab: (lab[b], 0, 0)),
                pl.BlockSpec((1, 1), lambda b, lab: (0, 0), memory_space=pltpu.SMEM),
                pl.BlockSpec((_G,), lambda b, lab: (0,), memory_space=pltpu.SMEM),
            ],
            out_specs=pl.BlockSpec((1, _S, _D), lambda b, lab: (b, 0, 0)),
        ),
        out_shape=jax.ShapeDtypeStruct((_B, _S, _D), jnp.float32),
    )(labels, xa, bw, xamax, bwmax)
    return out


# speedup vs baseline: 1.8857x; 1.2168x over previous
import jax
import jax.numpy as jnp
from jax.experimental import pallas as pl
from jax.experimental.pallas import tpu as pltpu

_B, _S, _D, _G = 32, 256, 768, 8
_QMAX = 127.0
_EPS = 1e-8


def _route_xa_kernel(x_ref, a_ref, c_ref, xa_ref, lab_ref, xam_ref):
    b = pl.program_id(0)
    x = x_ref[0]
    m = jnp.mean(x, axis=0, keepdims=True)
    xc = x - m
    var = jnp.sum(xc * xc, axis=0, keepdims=True) / (_S - 1)
    stats = jnp.concatenate([m, jnp.sqrt(var)], axis=1)
    diff = stats - c_ref[...]
    d2 = jnp.sum(diff * diff, axis=1, keepdims=True)
    idx = jax.lax.broadcasted_iota(jnp.int32, (_G, 1), 0)
    lab = jnp.min(jnp.where(d2 == jnp.min(d2), idx, _G)).astype(jnp.int32)
    lab_ref[b] = lab
    xa = jnp.dot(x, a_ref[lab], preferred_element_type=jnp.float32)
    xa_ref[0] = xa
    mx = jnp.max(jnp.abs(xa))

    @pl.when(b == 0)
    def _():
        xam_ref[0] = mx

    @pl.when(b > 0)
    def _():
        xam_ref[0] = jnp.maximum(xam_ref[0], mx)


def _bw_final_kernel(lab_ref, xam_ref, b_ref, w_ref, xa_ref, out_ref,
                     bw_scr, bwm_scr, scale_scr):
    t = pl.program_id(0)

    @pl.when(t < _G)
    def _bw():
        bw = jnp.dot(b_ref[0], w_ref[...], preferred_element_type=jnp.float32)
        bw_scr[t] = bw
        bwm_scr[t] = jnp.max(jnp.abs(bw))

    @pl.when(t == _G)
    def _scales():
        scale_scr[0] = jnp.maximum(xam_ref[0] / _QMAX, _EPS)
        bm = jnp.float32(0.0)
        for g in range(_G):
            used = lab_ref[0] == g
            for i in range(1, _B):
                used = used | (lab_ref[i] == g)
            bm = jnp.maximum(bm, jnp.where(used, bwm_scr[g], 0.0))
        scale_scr[1] = jnp.maximum(bm / _QMAX, _EPS)

    @pl.when(t > _G)
    def _final():
        b = t - (_G + 1)
        sxa = scale_scr[0]
        sbw = scale_scr[1]
        qxa = jnp.round(xa_ref[0] * (1.0 / sxa)).astype(jnp.bfloat16)
        qbw = jnp.round(bw_scr[lab_ref[b]] * (1.0 / sbw)).astype(jnp.bfloat16)
        acc = jnp.dot(qxa, qbw, preferred_element_type=jnp.float32)
        out_ref[0] = acc * (sxa * sbw)


def kernel(X, W, A_stack, B_stack, centroids):
    xa, labels, xamax = pl.pallas_call(
        _route_xa_kernel,
        grid=(_B,),
        in_specs=[
            pl.BlockSpec((1, _S, _D), lambda b: (b, 0, 0)),
            pl.BlockSpec((_G, _D, _D), lambda b: (0, 0, 0)),
            pl.BlockSpec((_G, 2 * _D), lambda b: (0, 0)),
        ],
        out_specs=[
            pl.BlockSpec((1, _S, _D), lambda b: (b, 0, 0)),
            pl.BlockSpec((_B,), lambda b: (0,), memory_space=pltpu.SMEM),
            pl.BlockSpec((1,), lambda b: (0,), memory_space=pltpu.SMEM),
        ],
        out_shape=[
            jax.ShapeDtypeStruct((_B, _S, _D), jnp.float32),
            jax.ShapeDtypeStruct((_B,), jnp.int32),
            jax.ShapeDtypeStruct((1,), jnp.float32),
        ],
    )(X, A_stack, centroids)

    out = pl.pallas_call(
        _bw_final_kernel,
        grid=(_G + 1 + _B,),
        in_specs=[
            pl.BlockSpec((_B,), lambda t: (0,), memory_space=pltpu.SMEM),
            pl.BlockSpec((1,), lambda t: (0,), memory_space=pltpu.SMEM),
            pl.BlockSpec((1, _D, _D), lambda t: (jnp.minimum(t, _G - 1), 0, 0)),
            pl.BlockSpec((_D, _D), lambda t: (0, 0)),
            pl.BlockSpec(
                (1, _S, _D),
                lambda t: (jnp.clip(t - (_G + 1), 0, _B - 1), 0, 0),
            ),
        ],
        out_specs=pl.BlockSpec(
            (1, _S, _D), lambda t: (jnp.clip(t - (_G + 1), 0, _B - 1), 0, 0)
        ),
        out_shape=jax.ShapeDtypeStruct((_B, _S, _D), jnp.float32),
        scratch_shapes=[
            pltpu.VMEM((_G, _D, _D), jnp.float32),
            pltpu.SMEM((_G,), jnp.float32),
            pltpu.SMEM((2,), jnp.float32),
        ],
    )(labels, xamax, B_stack, W, xa)
    return out


# speedup vs baseline: 2.4155x; 1.2810x over previous
import jax
import jax.numpy as jnp
from jax.experimental import pallas as pl
from jax.experimental.pallas import tpu as pltpu

_B, _S, _D, _G = 32, 256, 768, 8
_N = 4
_NB = _B // _N
_QMAX = 127.0
_EPS = 1e-8


def _route_xa_kernel(x_ref, a_ref, c_ref, xa_ref, lab_ref, xam_ref):
    t = pl.program_id(0)
    for i in range(_N):
        x = x_ref[i]
        m = jnp.mean(x, axis=0, keepdims=True)
        xc = x - m
        var = jnp.sum(xc * xc, axis=0, keepdims=True) / (_S - 1)
        stats = jnp.concatenate([m, jnp.sqrt(var)], axis=1)
        diff = stats - c_ref[...]
        d2 = jnp.sum(diff * diff, axis=1, keepdims=True)
        idx = jax.lax.broadcasted_iota(jnp.int32, (_G, 1), 0)
        lab = jnp.min(jnp.where(d2 == jnp.min(d2), idx, _G)).astype(jnp.int32)
        lab_ref[t * _N + i] = lab
        xa = jnp.dot(x, a_ref[lab], preferred_element_type=jnp.float32)
        xa_ref[i] = xa
        mx = jnp.max(jnp.abs(xa))

        @pl.when((t == 0) & (i == 0))
        def _():
            xam_ref[0] = mx

        @pl.when((t > 0) | (i > 0))
        def _():
            xam_ref[0] = jnp.maximum(xam_ref[0], mx)


def _bw_final_kernel(lab_ref, xam_ref, b_ref, w_ref, xa_ref, out_ref,
                     bw_scr, bwm_scr, scale_scr):
    t = pl.program_id(0)

    @pl.when(t < _G)
    def _bw():
        bw = jnp.dot(b_ref[0], w_ref[...], preferred_element_type=jnp.float32)
        bw_scr[t] = bw
        bwm_scr[t] = jnp.max(jnp.abs(bw))

    @pl.when(t == _G)
    def _scales():
        scale_scr[0] = jnp.maximum(xam_ref[0] / _QMAX, _EPS)
        bm = jnp.float32(0.0)
        for g in range(_G):
            used = lab_ref[0] == g
            for i in range(1, _B):
                used = used | (lab_ref[i] == g)
            bm = jnp.maximum(bm, jnp.where(used, bwm_scr[g], 0.0))
        scale_scr[1] = jnp.maximum(bm / _QMAX, _EPS)

    @pl.when(t > _G)
    def _final():
        blk = t - (_G + 1)
        sxa = scale_scr[0]
        sbw = scale_scr[1]
        for i in range(_N):
            qxa = jnp.round(xa_ref[i] * (1.0 / sxa)).astype(jnp.bfloat16)
            qbw = jnp.round(
                bw_scr[lab_ref[blk * _N + i]] * (1.0 / sbw)
            ).astype(jnp.bfloat16)
            acc = jnp.dot(qxa, qbw, preferred_element_type=jnp.float32)
            out_ref[i] = acc * (sxa * sbw)


def kernel(X, W, A_stack, B_stack, centroids):
    stream = pl.Buffered(buffer_count=2)
    xa, labels, xamax = pl.pallas_call(
        _route_xa_kernel,
        grid=(_NB,),
        in_specs=[
            pl.BlockSpec((_N, _S, _D), lambda t: (t, 0, 0), pipeline_mode=stream),
            pl.BlockSpec((_G, _D, _D), lambda t: (0, 0, 0)),
            pl.BlockSpec((_G, 2 * _D), lambda t: (0, 0)),
        ],
        out_specs=[
            pl.BlockSpec((_N, _S, _D), lambda t: (t, 0, 0), pipeline_mode=stream),
            pl.BlockSpec((_B,), lambda t: (0,), memory_space=pltpu.SMEM),
            pl.BlockSpec((1,), lambda t: (0,), memory_space=pltpu.SMEM),
        ],
        out_shape=[
            jax.ShapeDtypeStruct((_B, _S, _D), jnp.float32),
            jax.ShapeDtypeStruct((_B,), jnp.int32),
            jax.ShapeDtypeStruct((1,), jnp.float32),
        ],
    )(X, A_stack, centroids)

    out = pl.pallas_call(
        _bw_final_kernel,
        grid=(_G + 1 + _NB,),
        in_specs=[
            pl.BlockSpec((_B,), lambda t: (0,), memory_space=pltpu.SMEM),
            pl.BlockSpec((1,), lambda t: (0,), memory_space=pltpu.SMEM),
            pl.BlockSpec((1, _D, _D), lambda t: (jnp.minimum(t, _G - 1), 0, 0)),
            pl.BlockSpec((_D, _D), lambda t: (0, 0)),
            pl.BlockSpec(
                (_N, _S, _D),
                lambda t: (jnp.clip(t - (_G + 1), 0, _NB - 1), 0, 0),
                pipeline_mode=stream,
            ),
        ],
        out_specs=pl.BlockSpec(
            (_N, _S, _D),
            lambda t: (jnp.clip(t - (_G + 1), 0, _NB - 1), 0, 0),
            pipeline_mode=stream,
        ),
        out_shape=jax.ShapeDtypeStruct((_B, _S, _D), jnp.float32),
        scratch_shapes=[
            pltpu.VMEM((_G, _D, _D), jnp.float32),
            pltpu.SMEM((_G,), jnp.float32),
            pltpu.SMEM((2,), jnp.float32),
        ],
    )(labels, xamax, B_stack, W, xa)
    return out


# speedup vs baseline: 2.4311x; 1.0064x over previous
import jax
import jax.numpy as jnp
from jax.experimental import pallas as pl
from jax.experimental.pallas import tpu as pltpu

_B, _S, _D, _G = 32, 256, 768, 8
_N = 8
_NB = _B // _N
_QMAX = 127.0
_EPS = 1e-8


def _route_xa_kernel(x_ref, a_ref, c_ref, xa_ref, lab_ref, xam_ref):
    t = pl.program_id(0)
    for i in range(_N):
        x = x_ref[i]
        m = jnp.mean(x, axis=0, keepdims=True)
        xc = x - m
        var = jnp.sum(xc * xc, axis=0, keepdims=True) / (_S - 1)
        stats = jnp.concatenate([m, jnp.sqrt(var)], axis=1)
        diff = stats - c_ref[...]
        d2 = jnp.sum(diff * diff, axis=1, keepdims=True)
        idx = jax.lax.broadcasted_iota(jnp.int32, (_G, 1), 0)
        lab = jnp.min(jnp.where(d2 == jnp.min(d2), idx, _G)).astype(jnp.int32)
        lab_ref[t * _N + i] = lab
        xa = jnp.dot(x, a_ref[lab], preferred_element_type=jnp.float32)
        xa_ref[i] = xa
        mx = jnp.max(jnp.abs(xa))

        @pl.when((t == 0) & (i == 0))
        def _():
            xam_ref[0] = mx

        @pl.when((t > 0) | (i > 0))
        def _():
            xam_ref[0] = jnp.maximum(xam_ref[0], mx)


def _bw_final_kernel(lab_ref, xam_ref, b_ref, w_ref, xa_ref, out_ref,
                     bw_scr, bwm_scr, scale_scr):
    t = pl.program_id(0)

    @pl.when(t < _G)
    def _bw():
        bw = jnp.dot(b_ref[0], w_ref[...], preferred_element_type=jnp.float32)
        bw_scr[t] = bw
        bwm_scr[t] = jnp.max(jnp.abs(bw))

    @pl.when(t == _G)
    def _scales():
        scale_scr[0] = jnp.maximum(xam_ref[0] / _QMAX, _EPS)
        bm = jnp.float32(0.0)
        for g in range(_G):
            used = lab_ref[0] == g
            for i in range(1, _B):
                used = used | (lab_ref[i] == g)
            bm = jnp.maximum(bm, jnp.where(used, bwm_scr[g], 0.0))
        scale_scr[1] = jnp.maximum(bm / _QMAX, _EPS)

    @pl.when(t > _G)
    def _final():
        blk = t - (_G + 1)
        sxa = scale_scr[0]
        sbw = scale_scr[1]
        for i in range(_N):
            qxa = jnp.round(xa_ref[i] * (1.0 / sxa)).astype(jnp.bfloat16)
            qbw = jnp.round(
                bw_scr[lab_ref[blk * _N + i]] * (1.0 / sbw)
            ).astype(jnp.bfloat16)
            acc = jnp.dot(qxa, qbw, preferred_element_type=jnp.float32)
            out_ref[i] = acc * (sxa * sbw)


def kernel(X, W, A_stack, B_stack, centroids):
    stream = pl.Buffered(buffer_count=2)
    xa, labels, xamax = pl.pallas_call(
        _route_xa_kernel,
        grid=(_NB,),
        in_specs=[
            pl.BlockSpec((_N, _S, _D), lambda t: (t, 0, 0), pipeline_mode=stream),
            pl.BlockSpec((_G, _D, _D), lambda t: (0, 0, 0)),
            pl.BlockSpec((_G, 2 * _D), lambda t: (0, 0)),
        ],
        out_specs=[
            pl.BlockSpec((_N, _S, _D), lambda t: (t, 0, 0), pipeline_mode=stream),
            pl.BlockSpec((_B,), lambda t: (0,), memory_space=pltpu.SMEM),
            pl.BlockSpec((1,), lambda t: (0,), memory_space=pltpu.SMEM),
        ],
        out_shape=[
            jax.ShapeDtypeStruct((_B, _S, _D), jnp.float32),
            jax.ShapeDtypeStruct((_B,), jnp.int32),
            jax.ShapeDtypeStruct((1,), jnp.float32),
        ],
    )(X, A_stack, centroids)

    out = pl.pallas_call(
        _bw_final_kernel,
        grid=(_G + 1 + _NB,),
        in_specs=[
            pl.BlockSpec((_B,), lambda t: (0,), memory_space=pltpu.SMEM),
            pl.BlockSpec((1,), lambda t: (0,), memory_space=pltpu.SMEM),
            pl.BlockSpec((1, _D, _D), lambda t: (jnp.minimum(t, _G - 1), 0, 0)),
            pl.BlockSpec((_D, _D), lambda t: (0, 0)),
            pl.BlockSpec(
                (_N, _S, _D),
                lambda t: (jnp.clip(t - (_G + 1), 0, _NB - 1), 0, 0),
                pipeline_mode=stream,
            ),
        ],
        out_specs=pl.BlockSpec(
            (_N, _S, _D),
            lambda t: (jnp.clip(t - (_G + 1), 0, _NB - 1), 0, 0),
            pipeline_mode=stream,
        ),
        out_shape=jax.ShapeDtypeStruct((_B, _S, _D), jnp.float32),
        scratch_shapes=[
            pltpu.VMEM((_G, _D, _D), jnp.float32),
            pltpu.SMEM((_G,), jnp.float32),
            pltpu.SMEM((2,), jnp.float32),
        ],
    )(labels, xamax, B_stack, W, xa)
    return out
